# baseline (speedup 1.0000x reference)
import functools

import jax
import jax.numpy as jnp
from jax import lax
from jax.experimental import pallas as pl
from jax.experimental.pallas import tpu as pltpu
from jax.experimental.pallas import tpu_sc as plsc

D_MODEL = 1024
NUM_CORES = 2
NUM_SUBCORES = 16
NW = NUM_CORES * NUM_SUBCORES
CHUNK = 16
GROUP = 4


def _build_gather(bsz, seq):
    batch = bsz * seq
    bpw = batch // NW
    nch = bpw // CHUNK
    wpr = seq // bpw
    mesh = plsc.VectorSubcoreMesh(core_axis_name="c", subcore_axis_name="s")

    @functools.partial(
        pl.kernel,
        out_type=jax.ShapeDtypeStruct((batch, D_MODEL), jnp.float32),
        mesh=mesh,
        scratch_types=[
            pltpu.VMEM((bpw,), jnp.int32),
            pltpu.VMEM((2, 2 * CHUNK, D_MODEL), jnp.float32),
            [pltpu.SemaphoreType.DMA] * 4,
            [pltpu.SemaphoreType.DMA] * 2,
        ],
    )
    def gather_kernel(idx_hbm, table_hbm, out_hbm, idx_v, buf, gsems, ssems):
        wid = lax.axis_index("s") * NUM_CORES + lax.axis_index("c")
        base = wid * bpw

        def gather_dst(pp, half):
            return buf.at[pp].at[pl.ds(half * CHUNK, CHUNK)]

        def start_gather(c, pp, half):
            pltpu.async_copy(
                table_hbm.at[idx_v.at[pl.ds(c * CHUNK, CHUNK)]],
                gather_dst(pp, half),
                gsems[2 * pp + half],
            )

        def wait_gather(c, pp, half):
            pltpu.make_async_copy(
                table_hbm.at[idx_v.at[pl.ds(c * CHUNK, CHUNK)]],
                gather_dst(pp, half),
                gsems[2 * pp + half],
            ).wait()

        def start_scatter(c0, pp):
            pltpu.async_copy(
                buf.at[pp], out_hbm.at[pl.ds(base + c0 * CHUNK, 2 * CHUNK)], ssems[pp]
            )

        def wait_scatter(c0, pp):
            pltpu.make_async_copy(
                buf.at[pp], out_hbm.at[pl.ds(base + c0 * CHUNK, 2 * CHUNK)], ssems[pp]
            ).wait()

        pltpu.sync_copy(
            idx_hbm.at[wid // wpr].at[pl.ds((wid % wpr) * bpw, bpw)], idx_v
        )
        start_gather(0, 0, 0)
        start_gather(1, 0, 1)

        @pl.loop(0, nch, step=GROUP)
        def _(g):
            wait_gather(g, 0, 0)
            wait_gather(g + 1, 0, 1)
            start_scatter(g, 0)

            @pl.when(g - 2 >= 0)
            def _():
                wait_scatter(g - 2, 1)

            @pl.when(g + 2 < nch)
            def _():
                start_gather(g + 2, 1, 0)
                start_gather(g + 3, 1, 1)

            wait_gather(g + 2, 1, 0)
            wait_gather(g + 3, 1, 1)
            start_scatter(g + 2, 1)
            wait_scatter(g, 0)

            @pl.when(g + 4 < nch)
            def _():
                start_gather(g + 4, 0, 0)
                start_gather(g + 5, 0, 1)

        wait_scatter(nch - 2, 1)

    return gather_kernel


def kernel(position_ids, pe_weight):
    bsz, seq = position_ids.shape
    out = _build_gather(bsz, seq)(position_ids.astype(jnp.int32), pe_weight)
    return out.reshape(bsz, seq, D_MODEL)


# speedup vs baseline: 1.0093x; 1.0093x over previous
import functools

import jax
import jax.numpy as jnp
from jax import lax
from jax.experimental import pallas as pl
from jax.experimental.pallas import tpu as pltpu
from jax.experimental.pallas import tpu_sc as plsc

D_MODEL = 1024
NUM_CORES = 2
NUM_SUBCORES = 16
NW = NUM_CORES * NUM_SUBCORES
CHUNK = 16
NBUF = 4
LA = 3


def _build_gather(bsz, seq):
    batch = bsz * seq
    bpw = batch // NW
    nch = bpw // CHUNK
    wpr = seq // bpw
    mesh = plsc.VectorSubcoreMesh(core_axis_name="c", subcore_axis_name="s")

    @functools.partial(
        pl.kernel,
        out_type=jax.ShapeDtypeStruct((batch, D_MODEL), jnp.float32),
        mesh=mesh,
        scratch_types=[
            pltpu.VMEM((bpw,), jnp.int32),
            pltpu.VMEM((NBUF, CHUNK, D_MODEL), jnp.float32),
            [pltpu.SemaphoreType.DMA] * NBUF,
            [pltpu.SemaphoreType.DMA] * NBUF,
        ],
    )
    def gather_kernel(idx_hbm, table_hbm, out_hbm, idx_v, buf, gsems, ssems):
        wid = lax.axis_index("s") * NUM_CORES + lax.axis_index("c")
        base = wid * bpw

        def start_gather(c, b):
            pltpu.async_copy(
                table_hbm.at[idx_v.at[pl.ds(c * CHUNK, CHUNK)]], buf.at[b], gsems[b]
            )

        def wait_gather(c, b):
            pltpu.make_async_copy(
                table_hbm.at[idx_v.at[pl.ds(c * CHUNK, CHUNK)]], buf.at[b], gsems[b]
            ).wait()

        def start_scatter(c, b):
            pltpu.async_copy(
                buf.at[b], out_hbm.at[pl.ds(base + c * CHUNK, CHUNK)], ssems[b]
            )

        def wait_scatter(c, b):
            pltpu.make_async_copy(
                buf.at[b], out_hbm.at[pl.ds(base + c * CHUNK, CHUNK)], ssems[b]
            ).wait()

        pltpu.sync_copy(
            idx_hbm.at[wid // wpr].at[pl.ds((wid % wpr) * bpw, bpw)], idx_v
        )
        for b in range(LA):
            start_gather(b, b)

        @pl.loop(0, nch, step=NBUF)
        def _(g):
            for b in range(NBUF):
                c = g + b
                wait_gather(c, b)
                start_scatter(c, b)
                c2 = c + LA
                b2 = (b + LA) % NBUF

                @pl.when(jnp.logical_and(c2 - NBUF >= 0, c2 < nch))
                def _():
                    wait_scatter(c2 - NBUF, b2)

                @pl.when(c2 < nch)
                def _():
                    start_gather(c2, b2)

        for b in range(NBUF):
            wait_scatter(nch - NBUF + b, (nch - NBUF + b) % NBUF)

    return gather_kernel


def kernel(position_ids, pe_weight):
    bsz, seq = position_ids.shape
    out = _build_gather(bsz, seq)(position_ids.astype(jnp.int32), pe_weight)
    return out.reshape(bsz, seq, D_MODEL)


# speedup vs baseline: 1.0106x; 1.0013x over previous
import functools

import jax
import jax.numpy as jnp
from jax import lax
from jax.experimental import pallas as pl
from jax.experimental.pallas import tpu as pltpu
from jax.experimental.pallas import tpu_sc as plsc

D_MODEL = 1024
NUM_CORES = 2
NUM_SUBCORES = 16
NW = NUM_CORES * NUM_SUBCORES
CHUNK = 16
NBUF = 4
LA = 2


def _build_gather(bsz, seq):
    batch = bsz * seq
    bpw = batch // NW
    nch = bpw // CHUNK
    wpr = seq // bpw
    mesh = plsc.VectorSubcoreMesh(core_axis_name="c", subcore_axis_name="s")

    @functools.partial(
        pl.kernel,
        out_type=jax.ShapeDtypeStruct((batch, D_MODEL), jnp.float32),
        mesh=mesh,
        scratch_types=[
            pltpu.VMEM((bpw,), jnp.int32),
            pltpu.VMEM((NBUF, CHUNK, D_MODEL), jnp.float32),
            [pltpu.SemaphoreType.DMA] * NBUF,
            [pltpu.SemaphoreType.DMA] * NBUF,
        ],
    )
    def gather_kernel(idx_hbm, table_hbm, out_hbm, idx_v, buf, gsems, ssems):
        wid = lax.axis_index("s") * NUM_CORES + lax.axis_index("c")
        base = wid * bpw

        def start_gather(c, b):
            pltpu.async_copy(
                table_hbm.at[idx_v.at[pl.ds(c * CHUNK, CHUNK)]], buf.at[b], gsems[b]
            )

        def wait_gather(c, b):
            pltpu.make_async_copy(
                table_hbm.at[idx_v.at[pl.ds(c * CHUNK, CHUNK)]], buf.at[b], gsems[b]
            ).wait()

        def start_scatter(c, b):
            pltpu.async_copy(
                buf.at[b], out_hbm.at[pl.ds(base + c * CHUNK, CHUNK)], ssems[b]
            )

        def wait_scatter(c, b):
            pltpu.make_async_copy(
                buf.at[b], out_hbm.at[pl.ds(base + c * CHUNK, CHUNK)], ssems[b]
            ).wait()

        pltpu.sync_copy(
            idx_hbm.at[wid // wpr].at[pl.ds((wid % wpr) * bpw, bpw)], idx_v
        )
        for b in range(LA):
            start_gather(b, b)

        @pl.loop(0, nch, step=NBUF)
        def _(g):
            for b in range(NBUF):
                c = g + b
                wait_gather(c, b)
                start_scatter(c, b)
                c2 = c + LA
                b2 = (b + LA) % NBUF

                @pl.when(jnp.logical_and(c2 - NBUF >= 0, c2 < nch))
                def _():
                    wait_scatter(c2 - NBUF, b2)

                @pl.when(c2 < nch)
                def _():
                    start_gather(c2, b2)

        for b in range(NBUF):
            wait_scatter(nch - NBUF + b, (nch - NBUF + b) % NBUF)

    return gather_kernel


def kernel(position_ids, pe_weight):
    bsz, seq = position_ids.shape
    out = _build_gather(bsz, seq)(position_ids.astype(jnp.int32), pe_weight)
    return out.reshape(bsz, seq, D_MODEL)
